# baseline (speedup 1.0000x reference)
import functools

import jax
import jax.numpy as jnp
from jax import lax
from jax.experimental import pallas as pl
from jax.experimental.pallas import tpu as pltpu
from jax.experimental.pallas import tpu_sc as plsc

VOCAB = 100000
EMBED_DIM = 128
BATCH = 4096
SEQ_LEN = 200
N_IDX = BATCH * SEQ_LEN

_info = plsc.get_sparse_core_info()
_NC, _NS = _info.num_cores, _info.num_subcores
_NW = _NC * _NS
_PER_W = N_IDX // _NW
_CHUNK = 128
_N_CHUNKS = _PER_W // _CHUNK
_NBUF = 5
_N_OUTER = _N_CHUNKS // _NBUF

_mesh = plsc.VectorSubcoreMesh(core_axis_name="c", subcore_axis_name="s")


@functools.partial(
    pl.kernel,
    mesh=_mesh,
    out_type=jax.ShapeDtypeStruct((N_IDX, EMBED_DIM), jnp.float32),
    scratch_types=[
        pltpu.VMEM((_PER_W,), jnp.int32),
        pltpu.VMEM((_NBUF, _CHUNK, EMBED_DIM), jnp.float32),
        pltpu.SemaphoreType.DMA,
        pltpu.SemaphoreType.DMA((_NBUF,)),
        pltpu.SemaphoreType.DMA((_NBUF,)),
    ],
)
def _gather_kernel(idx_hbm, table_hbm, out_hbm, idx_v, rows_v, isem, gsem, osem):
    wid = lax.axis_index("s") * _NC + lax.axis_index("c")
    base = wid * _PER_W

    pltpu.async_copy(idx_hbm.at[pl.ds(base, _PER_W)], idx_v, isem).wait()

    def gather_args(g, b):
        return (
            table_hbm.at[idx_v.at[pl.ds(g * _CHUNK, _CHUNK)]],
            rows_v.at[b],
            gsem.at[b],
        )

    def put_args(g, b):
        return (
            rows_v.at[b],
            out_hbm.at[pl.ds(base + g * _CHUNK, _CHUNK)],
            osem.at[b],
        )

    for b in range(_NBUF):
        pltpu.async_copy(*gather_args(b, b))

    def outer(o, carry):
        for b in range(_NBUF):
            g = o * _NBUF + b
            pltpu.make_async_copy(*gather_args(g, b)).wait()
            pltpu.async_copy(*put_args(g, b))

        @pl.when(o < _N_OUTER - 1)
        def _():
            for b in range(_NBUF):
                pltpu.make_async_copy(*put_args(o * _NBUF + b, b)).wait()
                pltpu.async_copy(*gather_args((o + 1) * _NBUF + b, b))

        return carry

    lax.fori_loop(0, _N_OUTER, outer, 0)

    for b in range(_NBUF):
        pltpu.make_async_copy(*put_args((_N_OUTER - 1) * _NBUF + b, b)).wait()


def kernel(x, word_vectors):
    idx = x.reshape(-1).astype(jnp.int32)
    out = _gather_kernel(idx, word_vectors)
    return out.reshape(BATCH, SEQ_LEN, EMBED_DIM)


# speedup vs baseline: 1.0131x; 1.0131x over previous
import functools

import jax
import jax.numpy as jnp
from jax import lax
from jax.experimental import pallas as pl
from jax.experimental.pallas import tpu as pltpu
from jax.experimental.pallas import tpu_sc as plsc

VOCAB = 100000
EMBED_DIM = 128
BATCH = 4096
SEQ_LEN = 200
N_IDX = BATCH * SEQ_LEN

_info = plsc.get_sparse_core_info()
_NC, _NS = _info.num_cores, _info.num_subcores
_NW = _NC * _NS
_PER_W = N_IDX // _NW
_CHUNK = 128
_N_CHUNKS = _PER_W // _CHUNK
_NBUF = 4
_SKEW = 2
_N_OUTER = _N_CHUNKS // _NBUF

_mesh = plsc.VectorSubcoreMesh(core_axis_name="c", subcore_axis_name="s")


@functools.partial(
    pl.kernel,
    mesh=_mesh,
    out_type=jax.ShapeDtypeStruct((N_IDX, EMBED_DIM), jnp.float32),
    scratch_types=[
        pltpu.VMEM((_PER_W,), jnp.int32),
        pltpu.VMEM((_NBUF, _CHUNK, EMBED_DIM), jnp.float32),
        pltpu.SemaphoreType.DMA,
        pltpu.SemaphoreType.DMA((_NBUF,)),
        pltpu.SemaphoreType.DMA((_NBUF,)),
    ],
)
def _gather_kernel(idx_hbm, table_hbm, out_hbm, idx_v, rows_v, isem, gsem, osem):
    wid = lax.axis_index("s") * _NC + lax.axis_index("c")
    base = wid * _PER_W

    pltpu.async_copy(idx_hbm.at[pl.ds(base, _PER_W)], idx_v, isem).wait()

    def gather_args(g, b):
        return (
            table_hbm.at[idx_v.at[pl.ds(g * _CHUNK, _CHUNK)]],
            rows_v.at[b],
            gsem.at[b],
        )

    def put_args(g, b):
        return (
            rows_v.at[b],
            out_hbm.at[pl.ds(base + g * _CHUNK, _CHUNK)],
            osem.at[b],
        )

    for g in range(_SKEW):
        pltpu.async_copy(*gather_args(g, g % _NBUF))
    for b in range(_NBUF):
        g = b
        pltpu.make_async_copy(*gather_args(g, b)).wait()
        pltpu.async_copy(*put_args(g, b))
        if g - _SKEW >= 0:
            pltpu.make_async_copy(*put_args(g - _SKEW, (g - _SKEW) % _NBUF)).wait()
        pltpu.async_copy(*gather_args(g + _SKEW, (g + _SKEW) % _NBUF))

    def outer(o, carry):
        for b in range(_NBUF):
            g = o * _NBUF + b
            pltpu.make_async_copy(*gather_args(g, b)).wait()
            pltpu.async_copy(*put_args(g, b))
            pltpu.make_async_copy(*put_args(g - _SKEW, (b - _SKEW) % _NBUF)).wait()
            pltpu.async_copy(*gather_args(g + _SKEW, (b + _SKEW) % _NBUF))
        return carry

    lax.fori_loop(1, _N_OUTER - 1, outer, 0)

    for b in range(_NBUF):
        g = (_N_OUTER - 1) * _NBUF + b
        pltpu.make_async_copy(*gather_args(g, b)).wait()
        pltpu.async_copy(*put_args(g, b))
        pltpu.make_async_copy(*put_args(g - _SKEW, (b - _SKEW) % _NBUF)).wait()
        if g + _SKEW < _N_CHUNKS:
            pltpu.async_copy(*gather_args(g + _SKEW, (b + _SKEW) % _NBUF))
    for g in range(_N_CHUNKS - _SKEW, _N_CHUNKS):
        pltpu.make_async_copy(*put_args(g, g % _NBUF)).wait()


def kernel(x, word_vectors):
    idx = x.reshape(-1).astype(jnp.int32)
    out = _gather_kernel(idx, word_vectors)
    return out.reshape(BATCH, SEQ_LEN, EMBED_DIM)
